# baseline (speedup 1.0000x reference)
import functools

import jax
import jax.numpy as jnp
from jax import lax
from jax.experimental import pallas as pl
from jax.experimental.pallas import tpu as pltpu
from jax.experimental.pallas import tpu_sc as plsc

NC = 2
NS = 16
L = 16
NW = NC * NS


def _edge_pass(ht, a_s, a_d, src, dst, *, interpret=False):
    H, N = ht.shape
    E = src.shape[0]
    CH1 = 2000 if (E // NS) % 2000 == 0 else E // NS
    CH = 8000 if E % (2 * 8000) == 0 else E // NS
    assert H == 2 * NW
    assert (E // NS) % CH1 == 0 and CH1 % L == 0 and N % L == 0
    assert E % CH == 0 and (E // CH) % 2 == 0 and CH % L == 0 and CH >= CH1

    mesh = plsc.VectorSubcoreMesh(core_axis_name="c", subcore_axis_name="s", num_cores=NC, num_subcores=NS)

    @functools.partial(
        pl.kernel,
        mesh=mesh,
        out_type=(
            jax.ShapeDtypeStruct((H * N,), jnp.float32),
            jax.ShapeDtypeStruct((NW * N,), jnp.float32),
            jax.ShapeDtypeStruct((NC * E,), jnp.float32),
        ),
        scratch_types=[
            pltpu.VMEM((N,), jnp.float32),
            pltpu.VMEM((N,), jnp.float32),
            pltpu.VMEM((N,), jnp.float32),
            pltpu.VMEM((N,), jnp.float32),
            pltpu.VMEM((N,), jnp.float32),
            pltpu.VMEM((N,), jnp.float32),
            pltpu.VMEM((N,), jnp.float32),
            pltpu.VMEM((CH,), jnp.int32),
            pltpu.VMEM((CH,), jnp.int32),
            pltpu.VMEM((CH,), jnp.float32),
            pltpu.VMEM((CH,), jnp.int32),
            pltpu.VMEM((CH,), jnp.int32),
            pltpu.VMEM((CH,), jnp.float32),
            pltpu.SemaphoreType.DMA,
            pltpu.SemaphoreType.DMA,
        ],
        compiler_params=pltpu.CompilerParams(use_tc_tiling_on_sc=False, needs_layout_passes=False),
        interpret=interpret,
    )
    def body(ht_hbm, as_hbm, ad_hbm, src_hbm, dst_hbm,
             acc_hbm, denp_hbm, w_hbm,
             as_l, ad_l, h0, h1, acc0, acc1, den,
             sbuf, dbuf, wbuf, sbuf1, dbuf1, wbuf1, sem0, sem1):
        c = lax.axis_index("c")
        s = lax.axis_index("s")
        wid = c * NS + s
        f0 = wid * 2

        pltpu.sync_copy(as_hbm, as_l)
        pltpu.sync_copy(ad_hbm, ad_l)
        pltpu.sync_copy(ht_hbm.at[pl.ds(f0 * N, N)], h0)
        pltpu.sync_copy(ht_hbm.at[pl.ds((f0 + 1) * N, N)], h1)

        zero16 = jnp.zeros((L,), jnp.float32)

        def zbody(i, _):
            acc0[pl.ds(i * L, L)] = zero16
            acc1[pl.ds(i * L, L)] = zero16
            den[pl.ds(i * L, L)] = zero16
            return _
        lax.fori_loop(0, N // L, zbody, None)

        base_s = s * (E // NS)

        def p1_chunk(t, _):
            base = base_s + t * CH1
            pltpu.sync_copy(src_hbm.at[pl.ds(base, CH1)], sbuf.at[pl.ds(0, CH1)])
            pltpu.sync_copy(dst_hbm.at[pl.ds(base, CH1)], dbuf.at[pl.ds(0, CH1)])

            @plsc.parallel_loop(0, CH1 // L, unroll=4)
            def p1_g(g):
                s16 = sbuf[pl.ds(g * L, L)]
                d16 = dbuf[pl.ds(g * L, L)]
                e = plsc.load_gather(as_l, [s16]) + plsc.load_gather(ad_l, [d16])
                e = jnp.where(e >= 0.0, e, 0.2 * e)
                w = jnp.exp(e)
                wbuf[pl.ds(g * L, L)] = w
                plsc.addupdate_scatter(den, [d16], w)
            pltpu.sync_copy(wbuf.at[pl.ds(0, CH1)],
                            w_hbm.at[pl.ds(c * E + base, CH1)])
            return _
        lax.fori_loop(0, (E // NS) // CH1, p1_chunk, None)
        pltpu.sync_copy(den, denp_hbm.at[pl.ds(wid * N, N)])
        plsc.subcore_barrier()

        def issue(chunk, sb, db, wb, sem):
            base = chunk * CH
            pltpu.async_copy(src_hbm.at[pl.ds(base, CH)], sb, sem)
            pltpu.async_copy(dst_hbm.at[pl.ds(base, CH)], db, sem)
            pltpu.async_copy(w_hbm.at[pl.ds(c * E + base, CH)], wb, sem)

        def drain(chunk, sb, db, wb, sem):
            base = chunk * CH
            pltpu.make_async_copy(src_hbm.at[pl.ds(base, CH)], sb, sem).wait()
            pltpu.make_async_copy(dst_hbm.at[pl.ds(base, CH)], db, sem).wait()
            pltpu.make_async_copy(
                w_hbm.at[pl.ds(c * E + base, CH)], wb, sem).wait()

        def compute(sb, db, wb):
            @plsc.parallel_loop(0, CH // L, unroll=8)
            def p2_g(g):
                s16 = sb[pl.ds(g * L, L)]
                d16 = db[pl.ds(g * L, L)]
                w16 = wb[pl.ds(g * L, L)]
                v0 = plsc.load_gather(h0, [s16]) * w16
                plsc.addupdate_scatter(acc0, [d16], v0)
                v1 = plsc.load_gather(h1, [s16]) * w16
                plsc.addupdate_scatter(acc1, [d16], v1)

        nch = E // CH
        issue(0, sbuf, dbuf, wbuf, sem0)

        def p2_pair(t, carry):
            c0 = 2 * t
            drain(c0, sbuf, dbuf, wbuf, sem0)
            issue(c0 + 1, sbuf1, dbuf1, wbuf1, sem1)
            compute(sbuf, dbuf, wbuf)
            drain(c0 + 1, sbuf1, dbuf1, wbuf1, sem1)

            @pl.when(t + 1 < nch // 2)
            def _issue_next():
                issue(c0 + 2, sbuf, dbuf, wbuf, sem0)
            compute(sbuf1, dbuf1, wbuf1)
            return carry
        lax.fori_loop(0, nch // 2, p2_pair, None)

        pltpu.sync_copy(acc0, acc_hbm.at[pl.ds(f0 * N, N)])
        pltpu.sync_copy(acc1, acc_hbm.at[pl.ds((f0 + 1) * N, N)])

    return body(ht.reshape(H * N), a_s, a_d, src, dst)


def _rating_gather(pt, qt, u_idx, m_idx, *, interpret=False):
    H, N = pt.shape
    B = u_idx.shape[0]
    assert B % L == 0

    mesh = plsc.VectorSubcoreMesh(core_axis_name="c", subcore_axis_name="s", num_cores=NC, num_subcores=NS)

    @functools.partial(
        pl.kernel,
        mesh=mesh,
        out_type=jax.ShapeDtypeStruct((NW * B,), jnp.float32),
        scratch_types=[
            pltpu.VMEM((N,), jnp.float32),
            pltpu.VMEM((N,), jnp.float32),
            pltpu.VMEM((N,), jnp.float32),
            pltpu.VMEM((N,), jnp.float32),
            pltpu.VMEM((B,), jnp.int32),
            pltpu.VMEM((B,), jnp.int32),
            pltpu.VMEM((B,), jnp.float32),
        ],
        compiler_params=pltpu.CompilerParams(use_tc_tiling_on_sc=False, needs_layout_passes=False),
        interpret=interpret,
    )
    def body(pt_hbm, qt_hbm, u_hbm, m_hbm, out_hbm,
             p0, p1, q0, q1, ubuf, mbuf, obuf):
        c = lax.axis_index("c")
        s = lax.axis_index("s")
        wid = c * NS + s
        f0 = wid * 2

        pltpu.sync_copy(pt_hbm.at[pl.ds(f0 * N, N)], p0)
        pltpu.sync_copy(pt_hbm.at[pl.ds((f0 + 1) * N, N)], p1)
        pltpu.sync_copy(qt_hbm.at[pl.ds(f0 * N, N)], q0)
        pltpu.sync_copy(qt_hbm.at[pl.ds((f0 + 1) * N, N)], q1)
        pltpu.sync_copy(u_hbm, ubuf)
        pltpu.sync_copy(m_hbm, mbuf)

        @plsc.parallel_loop(0, B // L, unroll=8)
        def gbody(g):
            u16 = ubuf[pl.ds(g * L, L)]
            m16 = mbuf[pl.ds(g * L, L)]
            r = (plsc.load_gather(p0, [u16]) + plsc.load_gather(p1, [u16])
                 + plsc.load_gather(q0, [m16]) + plsc.load_gather(q1, [m16]))
            obuf[pl.ds(g * L, L)] = r
        pltpu.sync_copy(obuf, out_hbm.at[pl.ds(wid * B, B)])

    return body(pt.reshape(H * N), qt.reshape(H * N), u_idx, m_idx)


def _tc1_body(x_ref, w1t_ref, a1s_ref, a1d_ref, ht_ref, as_ref, ad_ref):
    ht = lax.dot_general(w1t_ref[...], x_ref[...],
                         (((1,), (1,)), ((), ())),
                         preferred_element_type=jnp.float32)
    ht_ref[...] = ht
    as_ref[...] = jnp.dot(a1s_ref[...], ht, preferred_element_type=jnp.float32)
    ad_ref[...] = jnp.dot(a1d_ref[...], ht, preferred_element_type=jnp.float32)


def _tc1(x, w1t, a1s, a1d):
    N = x.shape[0]
    H = w1t.shape[0]
    return pl.pallas_call(
        _tc1_body,
        out_shape=(
            jax.ShapeDtypeStruct((H, N), jnp.float32),
            jax.ShapeDtypeStruct((1, N), jnp.float32),
            jax.ShapeDtypeStruct((1, N), jnp.float32),
        ),
    )(x, w1t, a1s, a1d)


def _norm_next(accT, denP, ht, as_v, ad_v, b_col, w2t, a2s, a2d):
    def body(acc_ref, den_ref, ht_ref, as_ref, ad_ref, b_ref,
             w2t_ref, a2s_ref, a2d_ref, h2_ref, as2_ref, ad2_ref):
        e = as_ref[...] + ad_ref[...]
        wself = jnp.exp(jnp.where(e >= 0.0, e, 0.2 * e))
        den = jnp.sum(den_ref[...][:NS, :], axis=0, keepdims=True) + wself
        num = acc_ref[...] + wself * ht_ref[...]
        out1 = num / (den + 1e-16) + b_ref[...]
        r = jnp.maximum(out1, 0.0)
        h2 = lax.dot_general(w2t_ref[...], r, (((1,), (0,)), ((), ())),
                             preferred_element_type=jnp.float32)
        h2_ref[...] = h2
        as2_ref[...] = jnp.dot(a2s_ref[...], h2,
                               preferred_element_type=jnp.float32)
        ad2_ref[...] = jnp.dot(a2d_ref[...], h2,
                               preferred_element_type=jnp.float32)

    H, N = ht.shape
    return pl.pallas_call(
        body,
        out_shape=(
            jax.ShapeDtypeStruct((H, N), jnp.float32),
            jax.ShapeDtypeStruct((1, N), jnp.float32),
            jax.ShapeDtypeStruct((1, N), jnp.float32),
        ),
    )(accT, denP, ht, as_v, ad_v, b_col, w2t, a2s, a2d)


def _norm_scale(accT, denP, ht, as_v, ad_v, b_col, fwu, fwm):
    def body(acc_ref, den_ref, ht_ref, as_ref, ad_ref, b_ref,
             fwu_ref, fwm_ref, pt_ref, qt_ref):
        e = as_ref[...] + ad_ref[...]
        wself = jnp.exp(jnp.where(e >= 0.0, e, 0.2 * e))
        den = jnp.sum(den_ref[...][:NS, :], axis=0, keepdims=True) + wself
        num = acc_ref[...] + wself * ht_ref[...]
        out2 = num / (den + 1e-16) + b_ref[...]
        pt_ref[...] = out2 * fwu_ref[...]
        qt_ref[...] = out2 * fwm_ref[...]

    H, N = ht.shape
    return pl.pallas_call(
        body,
        out_shape=(
            jax.ShapeDtypeStruct((H, N), jnp.float32),
            jax.ShapeDtypeStruct((H, N), jnp.float32),
        ),
    )(accT, denP, ht, as_v, ad_v, b_col, fwu, fwm)


def _reduce_partials(partial, fcb):
    def body(p_ref, b_ref, o_ref):
        o_ref[...] = jnp.sum(p_ref[...], axis=0, keepdims=True) + b_ref[...]

    NWp, B = partial.shape
    return pl.pallas_call(
        body,
        out_shape=jax.ShapeDtypeStruct((1, B), jnp.float32),
    )(partial, fcb)


def kernel(x, edge_index, user_indices, movie_indices,
           W1, a1_src, a1_dst, b1, W2, a2_src, a2_dst, b2, fc_W, fc_b):
    N, F_in = x.shape
    H = W1.shape[1]
    E = edge_index.shape[1]
    B = user_indices.shape[0]

    src = edge_index[0]
    dst = edge_index[1]

    h1t, as1, ad1 = _tc1(x, W1.T, a1_src.reshape(1, H), a1_dst.reshape(1, H))
    acc1, denp1, _ = _edge_pass(h1t, as1.reshape(N), ad1.reshape(N), src, dst)
    acc1 = acc1.reshape(H, N)
    denp1 = denp1.reshape(NW, N)

    h2t, as2, ad2 = _norm_next(acc1, denp1, h1t, as1, ad1, b1.reshape(H, 1),
                               W2.T, a2_src.reshape(1, H), a2_dst.reshape(1, H))
    acc2, denp2, _ = _edge_pass(h2t, as2.reshape(N), ad2.reshape(N), src, dst)
    acc2 = acc2.reshape(H, N)
    denp2 = denp2.reshape(NW, N)

    pt, qt = _norm_scale(acc2, denp2, h2t, as2, ad2, b2.reshape(H, 1),
                         fc_W[:H].reshape(H, 1), fc_W[H:].reshape(H, 1))

    partial = _rating_gather(pt, qt, user_indices, movie_indices)
    out = _reduce_partials(partial.reshape(NW, B), fc_b.reshape(1, 1))
    return out.reshape(B, 1)


# speedup vs baseline: 1.0743x; 1.0743x over previous
import functools

import jax
import jax.numpy as jnp
from jax import lax
from jax.experimental import pallas as pl
from jax.experimental.pallas import tpu as pltpu
from jax.experimental.pallas import tpu_sc as plsc

NC = 2
NS = 16
L = 16
NW = NC * NS


def _edge_pass(ht, a_s, a_d, src, dst, *, interpret=False):
    H, N = ht.shape
    E = src.shape[0]
    EH = E // 2
    CH1 = 2000 if (E // NS) % 2000 == 0 else E // NS
    CH = 4000 if EH % (2 * 4000) == 0 else E // NS
    assert H == NC * (NS // 2) * 4 and NS % 2 == 0
    assert (E // NS) % CH1 == 0 and CH1 % L == 0 and N % L == 0
    assert EH % CH == 0 and (EH // CH) % 2 == 0 and CH % L == 0 and CH >= CH1

    mesh = plsc.VectorSubcoreMesh(core_axis_name="c", subcore_axis_name="s", num_cores=NC, num_subcores=NS)

    @functools.partial(
        pl.kernel,
        mesh=mesh,
        out_type=(
            jax.ShapeDtypeStruct((H * N,), jnp.float32),
            jax.ShapeDtypeStruct((NW * N,), jnp.float32),
            jax.ShapeDtypeStruct((NC * E,), jnp.float32),
            jax.ShapeDtypeStruct((H * N,), jnp.float32),
        ),
        scratch_types=[
            pltpu.VMEM((N,), jnp.float32),
            pltpu.VMEM((N,), jnp.float32),
            pltpu.VMEM((N,), jnp.float32),
            pltpu.VMEM((N,), jnp.float32),
            pltpu.VMEM((N,), jnp.float32),
            pltpu.VMEM((N,), jnp.float32),
            pltpu.VMEM((N,), jnp.float32),
            pltpu.VMEM((N,), jnp.float32),
            pltpu.VMEM((N,), jnp.float32),
            pltpu.VMEM((CH,), jnp.int32),
            pltpu.VMEM((CH,), jnp.int32),
            pltpu.VMEM((CH,), jnp.float32),
            pltpu.VMEM((CH,), jnp.int32),
            pltpu.VMEM((CH,), jnp.int32),
            pltpu.VMEM((CH,), jnp.float32),
            pltpu.SemaphoreType.DMA,
            pltpu.SemaphoreType.DMA,
        ],
        compiler_params=pltpu.CompilerParams(use_tc_tiling_on_sc=False, needs_layout_passes=False),
        interpret=interpret,
    )
    def body(ht_hbm, as_hbm, ad_hbm, src_hbm, dst_hbm,
             acc_hbm, denp_hbm, w_hbm, accp_hbm,
             h0, h1, t0, t1, acc0, acc1, acc2, acc3, den,
             sbuf, dbuf, wbuf, sbuf1, dbuf1, wbuf1, sem0, sem1):
        c = lax.axis_index("c")
        s = lax.axis_index("s")
        wid = c * NS + s
        half = s % 2
        base_f = (c * (NS // 2) + s // 2) * 4
        rows = [base_f + (k + 2 * half) % 4 for k in range(4)]

        pltpu.sync_copy(as_hbm, t0)
        pltpu.sync_copy(ad_hbm, t1)

        zero16 = jnp.zeros((L,), jnp.float32)

        def zbody(i, _):
            acc0[pl.ds(i * L, L)] = zero16
            acc1[pl.ds(i * L, L)] = zero16
            acc2[pl.ds(i * L, L)] = zero16
            acc3[pl.ds(i * L, L)] = zero16
            den[pl.ds(i * L, L)] = zero16
            return _
        lax.fori_loop(0, N // L, zbody, None)

        base_s = s * (E // NS)

        def p1_chunk(t, _):
            base = base_s + t * CH1
            pltpu.sync_copy(src_hbm.at[pl.ds(base, CH1)], sbuf.at[pl.ds(0, CH1)])
            pltpu.sync_copy(dst_hbm.at[pl.ds(base, CH1)], dbuf.at[pl.ds(0, CH1)])

            @plsc.parallel_loop(0, CH1 // L, unroll=4)
            def p1_g(g):
                s16 = sbuf[pl.ds(g * L, L)]
                d16 = dbuf[pl.ds(g * L, L)]
                e = plsc.load_gather(t0, [s16]) + plsc.load_gather(t1, [d16])
                e = jnp.where(e >= 0.0, e, 0.2 * e)
                w = jnp.exp(e)
                wbuf[pl.ds(g * L, L)] = w
                plsc.addupdate_scatter(den, [d16], w)
            pltpu.sync_copy(wbuf.at[pl.ds(0, CH1)],
                            w_hbm.at[pl.ds(c * E + base, CH1)])
            return _
        lax.fori_loop(0, (E // NS) // CH1, p1_chunk, None)
        pltpu.sync_copy(den, denp_hbm.at[pl.ds(wid * N, N)])
        plsc.subcore_barrier()

        hbufs = (h0, h1, t0, t1)
        accs = (acc0, acc1, acc2, acc3)
        for k in range(4):
            pltpu.sync_copy(ht_hbm.at[pl.ds(rows[k] * N, N)], hbufs[k])

        bo = half * EH

        def issue(chunk, sb, db, wb, sem):
            base = bo + chunk * CH
            pltpu.async_copy(src_hbm.at[pl.ds(base, CH)], sb, sem)
            pltpu.async_copy(dst_hbm.at[pl.ds(base, CH)], db, sem)
            pltpu.async_copy(w_hbm.at[pl.ds(c * E + base, CH)], wb, sem)

        def drain(chunk, sb, db, wb, sem):
            base = bo + chunk * CH
            pltpu.make_async_copy(src_hbm.at[pl.ds(base, CH)], sb, sem).wait()
            pltpu.make_async_copy(dst_hbm.at[pl.ds(base, CH)], db, sem).wait()
            pltpu.make_async_copy(
                w_hbm.at[pl.ds(c * E + base, CH)], wb, sem).wait()

        def compute(sb, db, wb):
            @plsc.parallel_loop(0, CH // L, unroll=8)
            def p2_g(g):
                s16 = sb[pl.ds(g * L, L)]
                d16 = db[pl.ds(g * L, L)]
                w16 = wb[pl.ds(g * L, L)]
                for k in range(4):
                    v = plsc.load_gather(hbufs[k], [s16]) * w16
                    plsc.addupdate_scatter(accs[k], [d16], v)

        nch = EH // CH
        issue(0, sbuf, dbuf, wbuf, sem0)

        def p2_pair(t, carry):
            c0 = 2 * t
            drain(c0, sbuf, dbuf, wbuf, sem0)
            issue(c0 + 1, sbuf1, dbuf1, wbuf1, sem1)
            compute(sbuf, dbuf, wbuf)
            drain(c0 + 1, sbuf1, dbuf1, wbuf1, sem1)

            @pl.when(t + 1 < nch // 2)
            def _issue_next():
                issue(c0 + 2, sbuf, dbuf, wbuf, sem0)
            compute(sbuf1, dbuf1, wbuf1)
            return carry
        lax.fori_loop(0, nch // 2, p2_pair, None)

        pltpu.sync_copy(acc2, accp_hbm.at[pl.ds(rows[2] * N, N)])
        pltpu.sync_copy(acc3, accp_hbm.at[pl.ds(rows[3] * N, N)])
        plsc.subcore_barrier()

        for k in range(2):
            pltpu.sync_copy(accp_hbm.at[pl.ds(rows[k] * N, N)], den)

            @plsc.parallel_loop(0, N // L, unroll=8)
            def red_g(g):
                accs[k][pl.ds(g * L, L)] = (
                    accs[k][pl.ds(g * L, L)] + den[pl.ds(g * L, L)])
            pltpu.sync_copy(accs[k], acc_hbm.at[pl.ds(rows[k] * N, N)])

    return body(ht.reshape(H * N), a_s, a_d, src, dst)[:3]


def _rating_gather(pt, qt, u_idx, m_idx, *, interpret=False):
    H, N = pt.shape
    B = u_idx.shape[0]
    assert B % L == 0

    mesh = plsc.VectorSubcoreMesh(core_axis_name="c", subcore_axis_name="s", num_cores=NC, num_subcores=NS)

    @functools.partial(
        pl.kernel,
        mesh=mesh,
        out_type=jax.ShapeDtypeStruct((NW * B,), jnp.float32),
        scratch_types=[
            pltpu.VMEM((N,), jnp.float32),
            pltpu.VMEM((N,), jnp.float32),
            pltpu.VMEM((N,), jnp.float32),
            pltpu.VMEM((N,), jnp.float32),
            pltpu.VMEM((B,), jnp.int32),
            pltpu.VMEM((B,), jnp.int32),
            pltpu.VMEM((B,), jnp.float32),
        ],
        compiler_params=pltpu.CompilerParams(use_tc_tiling_on_sc=False, needs_layout_passes=False),
        interpret=interpret,
    )
    def body(pt_hbm, qt_hbm, u_hbm, m_hbm, out_hbm,
             p0, p1, q0, q1, ubuf, mbuf, obuf):
        c = lax.axis_index("c")
        s = lax.axis_index("s")
        wid = c * NS + s
        f0 = wid * 2

        pltpu.sync_copy(pt_hbm.at[pl.ds(f0 * N, N)], p0)
        pltpu.sync_copy(pt_hbm.at[pl.ds((f0 + 1) * N, N)], p1)
        pltpu.sync_copy(qt_hbm.at[pl.ds(f0 * N, N)], q0)
        pltpu.sync_copy(qt_hbm.at[pl.ds((f0 + 1) * N, N)], q1)
        pltpu.sync_copy(u_hbm, ubuf)
        pltpu.sync_copy(m_hbm, mbuf)

        @plsc.parallel_loop(0, B // L, unroll=8)
        def gbody(g):
            u16 = ubuf[pl.ds(g * L, L)]
            m16 = mbuf[pl.ds(g * L, L)]
            r = (plsc.load_gather(p0, [u16]) + plsc.load_gather(p1, [u16])
                 + plsc.load_gather(q0, [m16]) + plsc.load_gather(q1, [m16]))
            obuf[pl.ds(g * L, L)] = r
        pltpu.sync_copy(obuf, out_hbm.at[pl.ds(wid * B, B)])

    return body(pt.reshape(H * N), qt.reshape(H * N), u_idx, m_idx)


def _tc1_body(x_ref, w1t_ref, a1s_ref, a1d_ref, ht_ref, as_ref, ad_ref):
    ht = lax.dot_general(w1t_ref[...], x_ref[...],
                         (((1,), (1,)), ((), ())),
                         preferred_element_type=jnp.float32)
    ht_ref[...] = ht
    as_ref[...] = jnp.dot(a1s_ref[...], ht, preferred_element_type=jnp.float32)
    ad_ref[...] = jnp.dot(a1d_ref[...], ht, preferred_element_type=jnp.float32)


def _tc1(x, w1t, a1s, a1d):
    N = x.shape[0]
    H = w1t.shape[0]
    return pl.pallas_call(
        _tc1_body,
        out_shape=(
            jax.ShapeDtypeStruct((H, N), jnp.float32),
            jax.ShapeDtypeStruct((1, N), jnp.float32),
            jax.ShapeDtypeStruct((1, N), jnp.float32),
        ),
    )(x, w1t, a1s, a1d)


def _norm_next(accT, denP, ht, as_v, ad_v, b_col, w2t, a2s, a2d):
    def body(acc_ref, den_ref, ht_ref, as_ref, ad_ref, b_ref,
             w2t_ref, a2s_ref, a2d_ref, h2_ref, as2_ref, ad2_ref):
        e = as_ref[...] + ad_ref[...]
        wself = jnp.exp(jnp.where(e >= 0.0, e, 0.2 * e))
        den = jnp.sum(den_ref[...][:NS, :], axis=0, keepdims=True) + wself
        num = acc_ref[...] + wself * ht_ref[...]
        out1 = num / (den + 1e-16) + b_ref[...]
        r = jnp.maximum(out1, 0.0)
        h2 = lax.dot_general(w2t_ref[...], r, (((1,), (0,)), ((), ())),
                             preferred_element_type=jnp.float32)
        h2_ref[...] = h2
        as2_ref[...] = jnp.dot(a2s_ref[...], h2,
                               preferred_element_type=jnp.float32)
        ad2_ref[...] = jnp.dot(a2d_ref[...], h2,
                               preferred_element_type=jnp.float32)

    H, N = ht.shape
    return pl.pallas_call(
        body,
        out_shape=(
            jax.ShapeDtypeStruct((H, N), jnp.float32),
            jax.ShapeDtypeStruct((1, N), jnp.float32),
            jax.ShapeDtypeStruct((1, N), jnp.float32),
        ),
    )(accT, denP, ht, as_v, ad_v, b_col, w2t, a2s, a2d)


def _norm_scale(accT, denP, ht, as_v, ad_v, b_col, fwu, fwm):
    def body(acc_ref, den_ref, ht_ref, as_ref, ad_ref, b_ref,
             fwu_ref, fwm_ref, pt_ref, qt_ref):
        e = as_ref[...] + ad_ref[...]
        wself = jnp.exp(jnp.where(e >= 0.0, e, 0.2 * e))
        den = jnp.sum(den_ref[...][:NS, :], axis=0, keepdims=True) + wself
        num = acc_ref[...] + wself * ht_ref[...]
        out2 = num / (den + 1e-16) + b_ref[...]
        pt_ref[...] = out2 * fwu_ref[...]
        qt_ref[...] = out2 * fwm_ref[...]

    H, N = ht.shape
    return pl.pallas_call(
        body,
        out_shape=(
            jax.ShapeDtypeStruct((H, N), jnp.float32),
            jax.ShapeDtypeStruct((H, N), jnp.float32),
        ),
    )(accT, denP, ht, as_v, ad_v, b_col, fwu, fwm)


def _reduce_partials(partial, fcb):
    def body(p_ref, b_ref, o_ref):
        o_ref[...] = jnp.sum(p_ref[...], axis=0, keepdims=True) + b_ref[...]

    NWp, B = partial.shape
    return pl.pallas_call(
        body,
        out_shape=jax.ShapeDtypeStruct((1, B), jnp.float32),
    )(partial, fcb)


def kernel(x, edge_index, user_indices, movie_indices,
           W1, a1_src, a1_dst, b1, W2, a2_src, a2_dst, b2, fc_W, fc_b):
    N, F_in = x.shape
    H = W1.shape[1]
    E = edge_index.shape[1]
    B = user_indices.shape[0]

    src = edge_index[0]
    dst = edge_index[1]

    h1t, as1, ad1 = _tc1(x, W1.T, a1_src.reshape(1, H), a1_dst.reshape(1, H))
    acc1, denp1, _ = _edge_pass(h1t, as1.reshape(N), ad1.reshape(N), src, dst)
    acc1 = acc1.reshape(H, N)
    denp1 = denp1.reshape(NW, N)

    h2t, as2, ad2 = _norm_next(acc1, denp1, h1t, as1, ad1, b1.reshape(H, 1),
                               W2.T, a2_src.reshape(1, H), a2_dst.reshape(1, H))
    acc2, denp2, _ = _edge_pass(h2t, as2.reshape(N), ad2.reshape(N), src, dst)
    acc2 = acc2.reshape(H, N)
    denp2 = denp2.reshape(NW, N)

    pt, qt = _norm_scale(acc2, denp2, h2t, as2, ad2, b2.reshape(H, 1),
                         fc_W[:H].reshape(H, 1), fc_W[H:].reshape(H, 1))

    partial = _rating_gather(pt, qt, user_indices, movie_indices)
    out = _reduce_partials(partial.reshape(NW, B), fc_b.reshape(1, 1))
    return out.reshape(B, 1)


# speedup vs baseline: 1.1582x; 1.0781x over previous
import functools

import jax
import jax.numpy as jnp
from jax import lax
from jax.experimental import pallas as pl
from jax.experimental.pallas import tpu as pltpu
from jax.experimental.pallas import tpu_sc as plsc

NC = 2
NS = 16
L = 16
NW = NC * NS


def _edge_pass(ht, a_s, a_d, src, dst, *, interpret=False):
    H, N = ht.shape
    E = src.shape[0]
    EH = E // 2
    CH1 = 2000 if (E // NS) % 2000 == 0 else E // NS
    CH = 4000 if EH % (2 * 4000) == 0 else E // NS
    assert H == NC * (NS // 2) * 4 and NS % 2 == 0
    assert (E // NS) % CH1 == 0 and CH1 % L == 0 and N % L == 0
    assert EH % CH == 0 and (EH // CH) % 2 == 0 and CH % L == 0 and CH >= CH1

    mesh = plsc.VectorSubcoreMesh(core_axis_name="c", subcore_axis_name="s", num_cores=NC, num_subcores=NS)

    @functools.partial(
        pl.kernel,
        mesh=mesh,
        out_type=(
            jax.ShapeDtypeStruct((H * N,), jnp.float32),
            jax.ShapeDtypeStruct((NW * N,), jnp.float32),
            jax.ShapeDtypeStruct((NC * E,), jnp.float32),
            jax.ShapeDtypeStruct((H * N,), jnp.float32),
        ),
        scratch_types=[
            pltpu.VMEM((N,), jnp.float32),
            pltpu.VMEM((N,), jnp.float32),
            pltpu.VMEM((N,), jnp.float32),
            pltpu.VMEM((N,), jnp.float32),
            pltpu.VMEM((N,), jnp.float32),
            pltpu.VMEM((N,), jnp.float32),
            pltpu.VMEM((N,), jnp.float32),
            pltpu.VMEM((N,), jnp.float32),
            pltpu.VMEM((N,), jnp.float32),
            pltpu.VMEM((CH,), jnp.int32),
            pltpu.VMEM((CH,), jnp.int32),
            pltpu.VMEM((CH,), jnp.float32),
            pltpu.VMEM((CH,), jnp.int32),
            pltpu.VMEM((CH,), jnp.int32),
            pltpu.VMEM((CH,), jnp.float32),
            pltpu.SemaphoreType.DMA,
            pltpu.SemaphoreType.DMA,
        ],
        compiler_params=pltpu.CompilerParams(use_tc_tiling_on_sc=False, needs_layout_passes=False),
        interpret=interpret,
    )
    def body(ht_hbm, as_hbm, ad_hbm, src_hbm, dst_hbm,
             acc_hbm, denp_hbm, w_hbm, accp_hbm,
             h0, h1, t0, t1, acc0, acc1, acc2, acc3, den,
             sbuf, dbuf, wbuf, sbuf1, dbuf1, wbuf1, sem0, sem1):
        c = lax.axis_index("c")
        s = lax.axis_index("s")
        wid = c * NS + s
        half = s % 2
        base_f = (c * (NS // 2) + s // 2) * 4
        rows = [base_f + (k + 2 * half) % 4 for k in range(4)]

        pltpu.sync_copy(as_hbm, t0)
        pltpu.sync_copy(ad_hbm, t1)

        zero16 = jnp.zeros((L,), jnp.float32)

        def zbody(i, _):
            acc0[pl.ds(i * L, L)] = zero16
            acc1[pl.ds(i * L, L)] = zero16
            acc2[pl.ds(i * L, L)] = zero16
            acc3[pl.ds(i * L, L)] = zero16
            den[pl.ds(i * L, L)] = zero16
            return _
        lax.fori_loop(0, N // L, zbody, None)

        base_s = s * (E // NS)

        def p1_chunk(t, _):
            base = base_s + t * CH1
            pltpu.sync_copy(src_hbm.at[pl.ds(base, CH1)], sbuf.at[pl.ds(0, CH1)])
            pltpu.sync_copy(dst_hbm.at[pl.ds(base, CH1)], dbuf.at[pl.ds(0, CH1)])

            @plsc.parallel_loop(0, CH1 // L, unroll=4)
            def p1_g(g):
                s16 = sbuf[pl.ds(g * L, L)]
                d16 = dbuf[pl.ds(g * L, L)]
                e = plsc.load_gather(t0, [s16]) + plsc.load_gather(t1, [d16])
                e = jnp.where(e >= 0.0, e, 0.2 * e)
                w = jnp.exp(e)
                wbuf[pl.ds(g * L, L)] = w
                plsc.addupdate_scatter(den, [d16], w)
            pltpu.sync_copy(wbuf.at[pl.ds(0, CH1)],
                            w_hbm.at[pl.ds(c * E + base, CH1)])
            return _
        lax.fori_loop(0, (E // NS) // CH1, p1_chunk, None)
        pltpu.sync_copy(den, denp_hbm.at[pl.ds(wid * N, N)])
        plsc.subcore_barrier()

        accs = (acc0, acc1, acc2, acc3)
        for k, hp in ((0, t0), (2, t1)):
            pltpu.sync_copy(ht_hbm.at[pl.ds(rows[k] * N, N)], h0)
            pltpu.sync_copy(ht_hbm.at[pl.ds(rows[k + 1] * N, N)], h1)

            @plsc.parallel_loop(0, N // L, unroll=8)
            def pk_g(g):
                a = h0[pl.ds(g * L, L)]
                b = h1[pl.ds(g * L, L)]
                packed = plsc.pack(a, b, format=plsc.PackFormat.INTERLEAVED)
                hp[pl.ds(g * L, L)] = plsc.bitcast(packed, jnp.float32)

        bo = half * EH

        def issue(chunk, sb, db, wb, sem):
            base = bo + chunk * CH
            pltpu.async_copy(src_hbm.at[pl.ds(base, CH)], sb, sem)
            pltpu.async_copy(dst_hbm.at[pl.ds(base, CH)], db, sem)
            pltpu.async_copy(w_hbm.at[pl.ds(c * E + base, CH)], wb, sem)

        def drain(chunk, sb, db, wb, sem):
            base = bo + chunk * CH
            pltpu.make_async_copy(src_hbm.at[pl.ds(base, CH)], sb, sem).wait()
            pltpu.make_async_copy(dst_hbm.at[pl.ds(base, CH)], db, sem).wait()
            pltpu.make_async_copy(
                w_hbm.at[pl.ds(c * E + base, CH)], wb, sem).wait()

        def compute(sb, db, wb):
            @plsc.parallel_loop(0, CH // L, unroll=8)
            def p2_g(g):
                s16 = sb[pl.ds(g * L, L)]
                d16 = db[pl.ds(g * L, L)]
                w16 = wb[pl.ds(g * L, L)]
                for k, hp in ((0, t0), (2, t1)):
                    g32 = plsc.load_gather(hp, [s16])
                    ab = plsc.bitcast(g32, jnp.bfloat16)
                    va, vb = plsc.unpack(
                        ab, format=plsc.PackFormat.INTERLEAVED)
                    plsc.addupdate_scatter(accs[k], [d16], va * w16)
                    plsc.addupdate_scatter(accs[k + 1], [d16], vb * w16)

        nch = EH // CH
        issue(0, sbuf, dbuf, wbuf, sem0)

        def p2_pair(t, carry):
            c0 = 2 * t
            drain(c0, sbuf, dbuf, wbuf, sem0)
            issue(c0 + 1, sbuf1, dbuf1, wbuf1, sem1)
            compute(sbuf, dbuf, wbuf)
            drain(c0 + 1, sbuf1, dbuf1, wbuf1, sem1)

            @pl.when(t + 1 < nch // 2)
            def _issue_next():
                issue(c0 + 2, sbuf, dbuf, wbuf, sem0)
            compute(sbuf1, dbuf1, wbuf1)
            return carry
        lax.fori_loop(0, nch // 2, p2_pair, None)

        pltpu.sync_copy(acc2, accp_hbm.at[pl.ds(rows[2] * N, N)])
        pltpu.sync_copy(acc3, accp_hbm.at[pl.ds(rows[3] * N, N)])
        plsc.subcore_barrier()

        for k in range(2):
            pltpu.sync_copy(accp_hbm.at[pl.ds(rows[k] * N, N)], den)

            @plsc.parallel_loop(0, N // L, unroll=8)
            def red_g(g):
                accs[k][pl.ds(g * L, L)] = (
                    accs[k][pl.ds(g * L, L)] + den[pl.ds(g * L, L)])
            pltpu.sync_copy(accs[k], acc_hbm.at[pl.ds(rows[k] * N, N)])

    return body(ht.reshape(H * N), a_s, a_d, src, dst)[:3]


def _rating_gather(pt, qt, u_idx, m_idx, *, interpret=False):
    H, N = pt.shape
    B = u_idx.shape[0]
    assert B % L == 0

    mesh = plsc.VectorSubcoreMesh(core_axis_name="c", subcore_axis_name="s", num_cores=NC, num_subcores=NS)

    @functools.partial(
        pl.kernel,
        mesh=mesh,
        out_type=jax.ShapeDtypeStruct((NW * B,), jnp.float32),
        scratch_types=[
            pltpu.VMEM((N,), jnp.float32),
            pltpu.VMEM((N,), jnp.float32),
            pltpu.VMEM((N,), jnp.float32),
            pltpu.VMEM((N,), jnp.float32),
            pltpu.VMEM((B,), jnp.int32),
            pltpu.VMEM((B,), jnp.int32),
            pltpu.VMEM((B,), jnp.float32),
        ],
        compiler_params=pltpu.CompilerParams(use_tc_tiling_on_sc=False, needs_layout_passes=False),
        interpret=interpret,
    )
    def body(pt_hbm, qt_hbm, u_hbm, m_hbm, out_hbm,
             p0, p1, q0, q1, ubuf, mbuf, obuf):
        c = lax.axis_index("c")
        s = lax.axis_index("s")
        wid = c * NS + s
        f0 = wid * 2

        pltpu.sync_copy(pt_hbm.at[pl.ds(f0 * N, N)], p0)
        pltpu.sync_copy(pt_hbm.at[pl.ds((f0 + 1) * N, N)], p1)
        pltpu.sync_copy(qt_hbm.at[pl.ds(f0 * N, N)], q0)
        pltpu.sync_copy(qt_hbm.at[pl.ds((f0 + 1) * N, N)], q1)
        pltpu.sync_copy(u_hbm, ubuf)
        pltpu.sync_copy(m_hbm, mbuf)

        @plsc.parallel_loop(0, B // L, unroll=8)
        def gbody(g):
            u16 = ubuf[pl.ds(g * L, L)]
            m16 = mbuf[pl.ds(g * L, L)]
            r = (plsc.load_gather(p0, [u16]) + plsc.load_gather(p1, [u16])
                 + plsc.load_gather(q0, [m16]) + plsc.load_gather(q1, [m16]))
            obuf[pl.ds(g * L, L)] = r
        pltpu.sync_copy(obuf, out_hbm.at[pl.ds(wid * B, B)])

    return body(pt.reshape(H * N), qt.reshape(H * N), u_idx, m_idx)


def _tc1_body(x_ref, w1t_ref, a1s_ref, a1d_ref, ht_ref, as_ref, ad_ref):
    ht = lax.dot_general(w1t_ref[...], x_ref[...],
                         (((1,), (1,)), ((), ())),
                         preferred_element_type=jnp.float32)
    ht_ref[...] = ht
    as_ref[...] = jnp.dot(a1s_ref[...], ht, preferred_element_type=jnp.float32)
    ad_ref[...] = jnp.dot(a1d_ref[...], ht, preferred_element_type=jnp.float32)


def _tc1(x, w1t, a1s, a1d):
    N = x.shape[0]
    H = w1t.shape[0]
    return pl.pallas_call(
        _tc1_body,
        out_shape=(
            jax.ShapeDtypeStruct((H, N), jnp.float32),
            jax.ShapeDtypeStruct((1, N), jnp.float32),
            jax.ShapeDtypeStruct((1, N), jnp.float32),
        ),
    )(x, w1t, a1s, a1d)


def _norm_next(accT, denP, ht, as_v, ad_v, b_col, w2t, a2s, a2d):
    def body(acc_ref, den_ref, ht_ref, as_ref, ad_ref, b_ref,
             w2t_ref, a2s_ref, a2d_ref, h2_ref, as2_ref, ad2_ref):
        e = as_ref[...] + ad_ref[...]
        wself = jnp.exp(jnp.where(e >= 0.0, e, 0.2 * e))
        den = jnp.sum(den_ref[...][:NS, :], axis=0, keepdims=True) + wself
        num = acc_ref[...] + wself * ht_ref[...]
        out1 = num / (den + 1e-16) + b_ref[...]
        r = jnp.maximum(out1, 0.0)
        h2 = lax.dot_general(w2t_ref[...], r, (((1,), (0,)), ((), ())),
                             preferred_element_type=jnp.float32)
        h2_ref[...] = h2
        as2_ref[...] = jnp.dot(a2s_ref[...], h2,
                               preferred_element_type=jnp.float32)
        ad2_ref[...] = jnp.dot(a2d_ref[...], h2,
                               preferred_element_type=jnp.float32)

    H, N = ht.shape
    return pl.pallas_call(
        body,
        out_shape=(
            jax.ShapeDtypeStruct((H, N), jnp.float32),
            jax.ShapeDtypeStruct((1, N), jnp.float32),
            jax.ShapeDtypeStruct((1, N), jnp.float32),
        ),
    )(accT, denP, ht, as_v, ad_v, b_col, w2t, a2s, a2d)


def _norm_scale(accT, denP, ht, as_v, ad_v, b_col, fwu, fwm):
    def body(acc_ref, den_ref, ht_ref, as_ref, ad_ref, b_ref,
             fwu_ref, fwm_ref, pt_ref, qt_ref):
        e = as_ref[...] + ad_ref[...]
        wself = jnp.exp(jnp.where(e >= 0.0, e, 0.2 * e))
        den = jnp.sum(den_ref[...][:NS, :], axis=0, keepdims=True) + wself
        num = acc_ref[...] + wself * ht_ref[...]
        out2 = num / (den + 1e-16) + b_ref[...]
        pt_ref[...] = out2 * fwu_ref[...]
        qt_ref[...] = out2 * fwm_ref[...]

    H, N = ht.shape
    return pl.pallas_call(
        body,
        out_shape=(
            jax.ShapeDtypeStruct((H, N), jnp.float32),
            jax.ShapeDtypeStruct((H, N), jnp.float32),
        ),
    )(accT, denP, ht, as_v, ad_v, b_col, fwu, fwm)


def _reduce_partials(partial, fcb):
    def body(p_ref, b_ref, o_ref):
        o_ref[...] = jnp.sum(p_ref[...], axis=0, keepdims=True) + b_ref[...]

    NWp, B = partial.shape
    return pl.pallas_call(
        body,
        out_shape=jax.ShapeDtypeStruct((1, B), jnp.float32),
    )(partial, fcb)


def kernel(x, edge_index, user_indices, movie_indices,
           W1, a1_src, a1_dst, b1, W2, a2_src, a2_dst, b2, fc_W, fc_b):
    N, F_in = x.shape
    H = W1.shape[1]
    E = edge_index.shape[1]
    B = user_indices.shape[0]

    src = edge_index[0]
    dst = edge_index[1]

    h1t, as1, ad1 = _tc1(x, W1.T, a1_src.reshape(1, H), a1_dst.reshape(1, H))
    acc1, denp1, _ = _edge_pass(h1t, as1.reshape(N), ad1.reshape(N), src, dst)
    acc1 = acc1.reshape(H, N)
    denp1 = denp1.reshape(NW, N)

    h2t, as2, ad2 = _norm_next(acc1, denp1, h1t, as1, ad1, b1.reshape(H, 1),
                               W2.T, a2_src.reshape(1, H), a2_dst.reshape(1, H))
    acc2, denp2, _ = _edge_pass(h2t, as2.reshape(N), ad2.reshape(N), src, dst)
    acc2 = acc2.reshape(H, N)
    denp2 = denp2.reshape(NW, N)

    pt, qt = _norm_scale(acc2, denp2, h2t, as2, ad2, b2.reshape(H, 1),
                         fc_W[:H].reshape(H, 1), fc_W[H:].reshape(H, 1))

    partial = _rating_gather(pt, qt, user_indices, movie_indices)
    out = _reduce_partials(partial.reshape(NW, B), fc_b.reshape(1, 1))
    return out.reshape(B, 1)


# speedup vs baseline: 1.2388x; 1.0696x over previous
import functools

import jax
import jax.numpy as jnp
from jax import lax
from jax.experimental import pallas as pl
from jax.experimental.pallas import tpu as pltpu
from jax.experimental.pallas import tpu_sc as plsc

NC = 2
NS = 16
L = 16
NW = NC * NS


def _edge_pass(ht, a_s, a_d, src, dst, *, interpret=False):
    H, N = ht.shape
    E = src.shape[0]
    EH = E // 2
    CH1 = 2000 if (E // NS) % 2000 == 0 else E // NS
    CH = 4000 if EH % (2 * 4000) == 0 else E // NS
    assert H == NC * (NS // 2) * 4 and NS % 2 == 0
    assert (E // NS) % CH1 == 0 and CH1 % L == 0 and N % L == 0
    assert EH % CH == 0 and (EH // CH) % 2 == 0 and CH % L == 0 and CH >= CH1

    mesh = plsc.VectorSubcoreMesh(core_axis_name="c", subcore_axis_name="s", num_cores=NC, num_subcores=NS)

    @functools.partial(
        pl.kernel,
        mesh=mesh,
        out_type=(
            jax.ShapeDtypeStruct((H * N,), jnp.float32),
            jax.ShapeDtypeStruct((NW * N,), jnp.float32),
            jax.ShapeDtypeStruct((NC * E,), jnp.float32),
            jax.ShapeDtypeStruct((H * N,), jnp.float32),
        ),
        scratch_types=[
            pltpu.VMEM((N,), jnp.float32),
            pltpu.VMEM((N,), jnp.float32),
            pltpu.VMEM((N,), jnp.float32),
            pltpu.VMEM((N,), jnp.float32),
            pltpu.VMEM((N,), jnp.float32),
            pltpu.VMEM((N,), jnp.float32),
            pltpu.VMEM((N,), jnp.float32),
            pltpu.VMEM((N,), jnp.float32),
            pltpu.VMEM((N,), jnp.float32),
            pltpu.VMEM((CH,), jnp.int32),
            pltpu.VMEM((CH,), jnp.int32),
            pltpu.VMEM((CH,), jnp.float32),
            pltpu.VMEM((CH,), jnp.int32),
            pltpu.VMEM((CH,), jnp.int32),
            pltpu.VMEM((CH,), jnp.float32),
            pltpu.SemaphoreType.DMA,
            pltpu.SemaphoreType.DMA,
            pltpu.SemaphoreType.DMA,
            pltpu.SemaphoreType.DMA,
        ],
        compiler_params=pltpu.CompilerParams(use_tc_tiling_on_sc=False, needs_layout_passes=False),
        interpret=interpret,
    )
    def body(ht_hbm, as_hbm, ad_hbm, src_hbm, dst_hbm,
             acc_hbm, denp_hbm, w_hbm, accp_hbm,
             h0, h1, t0, t1, acc0, acc1, acc2, acc3, den,
             sbuf, dbuf, wbuf, sbuf1, dbuf1, wbuf1,
             sem0, sem1, semw0, semw1):
        c = lax.axis_index("c")
        s = lax.axis_index("s")
        wid = c * NS + s
        half = s % 2
        base_f = (c * (NS // 2) + s // 2) * 4
        rows = [base_f + (k + 2 * half) % 4 for k in range(4)]

        pltpu.sync_copy(as_hbm, t0)
        pltpu.sync_copy(ad_hbm, t1)

        zero16 = jnp.zeros((L,), jnp.float32)

        def zbody(i, _):
            acc0[pl.ds(i * L, L)] = zero16
            acc1[pl.ds(i * L, L)] = zero16
            acc2[pl.ds(i * L, L)] = zero16
            acc3[pl.ds(i * L, L)] = zero16
            den[pl.ds(i * L, L)] = zero16
            return _
        lax.fori_loop(0, N // L, zbody, None)

        base_s = s * (E // NS)
        nch1 = (E // NS) // CH1
        assert nch1 % 2 == 0

        def p1_issue(t, sb, db, sem):
            base = base_s + t * CH1
            pltpu.async_copy(src_hbm.at[pl.ds(base, CH1)],
                             sb.at[pl.ds(0, CH1)], sem)
            pltpu.async_copy(dst_hbm.at[pl.ds(base, CH1)],
                             db.at[pl.ds(0, CH1)], sem)

        def p1_drain(t, sb, db, sem):
            base = base_s + t * CH1
            pltpu.make_async_copy(src_hbm.at[pl.ds(base, CH1)],
                                  sb.at[pl.ds(0, CH1)], sem).wait()
            pltpu.make_async_copy(dst_hbm.at[pl.ds(base, CH1)],
                                  db.at[pl.ds(0, CH1)], sem).wait()

        def p1_compute(t, sb, db, wb, wsem):
            @plsc.parallel_loop(0, CH1 // L, unroll=4)
            def p1_g(g):
                s16 = sb[pl.ds(g * L, L)]
                d16 = db[pl.ds(g * L, L)]
                e = plsc.load_gather(t0, [s16]) + plsc.load_gather(t1, [d16])
                e = jnp.where(e >= 0.0, e, 0.2 * e)
                w = jnp.exp(e)
                wb[pl.ds(g * L, L)] = w
                plsc.addupdate_scatter(den, [d16], w)
            base = base_s + t * CH1
            pltpu.async_copy(wb.at[pl.ds(0, CH1)],
                             w_hbm.at[pl.ds(c * E + base, CH1)], wsem)

        def p1_wdrain(t, wb, wsem):
            base = base_s + t * CH1
            pltpu.make_async_copy(
                wb.at[pl.ds(0, CH1)],
                w_hbm.at[pl.ds(c * E + base, CH1)], wsem).wait()

        p1_issue(0, sbuf, dbuf, sem0)

        def p1_pair(t, carry):
            c0 = 2 * t
            p1_drain(c0, sbuf, dbuf, sem0)
            p1_issue(c0 + 1, sbuf1, dbuf1, sem1)

            @pl.when(t > 0)
            def _wd0():
                p1_wdrain(c0 - 2, wbuf, semw0)
            p1_compute(c0, sbuf, dbuf, wbuf, semw0)
            p1_drain(c0 + 1, sbuf1, dbuf1, sem1)

            @pl.when(t + 1 < nch1 // 2)
            def _issue_next():
                p1_issue(c0 + 2, sbuf, dbuf, sem0)

            @pl.when(t > 0)
            def _wd1():
                p1_wdrain(c0 - 1, wbuf1, semw1)
            p1_compute(c0 + 1, sbuf1, dbuf1, wbuf1, semw1)
            return carry
        lax.fori_loop(0, nch1 // 2, p1_pair, None)
        p1_wdrain(nch1 - 2, wbuf, semw0)
        p1_wdrain(nch1 - 1, wbuf1, semw1)
        pltpu.sync_copy(den, denp_hbm.at[pl.ds(wid * N, N)])
        plsc.subcore_barrier()

        accs = (acc0, acc1, acc2, acc3)
        for k, hp in ((0, t0), (2, t1)):
            pltpu.sync_copy(ht_hbm.at[pl.ds(rows[k] * N, N)], h0)
            pltpu.sync_copy(ht_hbm.at[pl.ds(rows[k + 1] * N, N)], h1)

            @plsc.parallel_loop(0, N // L, unroll=8)
            def pk_g(g):
                a = h0[pl.ds(g * L, L)]
                b = h1[pl.ds(g * L, L)]
                packed = plsc.pack(a, b, format=plsc.PackFormat.INTERLEAVED)
                hp[pl.ds(g * L, L)] = plsc.bitcast(packed, jnp.float32)

        bo = half * EH

        def issue(chunk, sb, db, wb, sem):
            base = bo + chunk * CH
            pltpu.async_copy(src_hbm.at[pl.ds(base, CH)], sb, sem)
            pltpu.async_copy(dst_hbm.at[pl.ds(base, CH)], db, sem)
            pltpu.async_copy(w_hbm.at[pl.ds(c * E + base, CH)], wb, sem)

        def drain(chunk, sb, db, wb, sem):
            base = bo + chunk * CH
            pltpu.make_async_copy(src_hbm.at[pl.ds(base, CH)], sb, sem).wait()
            pltpu.make_async_copy(dst_hbm.at[pl.ds(base, CH)], db, sem).wait()
            pltpu.make_async_copy(
                w_hbm.at[pl.ds(c * E + base, CH)], wb, sem).wait()

        def compute(sb, db, wb):
            @plsc.parallel_loop(0, CH // L, unroll=8)
            def p2_g(g):
                s16 = sb[pl.ds(g * L, L)]
                d16 = db[pl.ds(g * L, L)]
                w16 = wb[pl.ds(g * L, L)]
                for k, hp in ((0, t0), (2, t1)):
                    g32 = plsc.load_gather(hp, [s16])
                    ab = plsc.bitcast(g32, jnp.bfloat16)
                    va, vb = plsc.unpack(
                        ab, format=plsc.PackFormat.INTERLEAVED)
                    plsc.addupdate_scatter(accs[k], [d16], va * w16)
                    plsc.addupdate_scatter(accs[k + 1], [d16], vb * w16)

        nch = EH // CH
        issue(0, sbuf, dbuf, wbuf, sem0)

        def p2_pair(t, carry):
            c0 = 2 * t
            drain(c0, sbuf, dbuf, wbuf, sem0)
            issue(c0 + 1, sbuf1, dbuf1, wbuf1, sem1)
            compute(sbuf, dbuf, wbuf)
            drain(c0 + 1, sbuf1, dbuf1, wbuf1, sem1)

            @pl.when(t + 1 < nch // 2)
            def _issue_next():
                issue(c0 + 2, sbuf, dbuf, wbuf, sem0)
            compute(sbuf1, dbuf1, wbuf1)
            return carry
        lax.fori_loop(0, nch // 2, p2_pair, None)

        pltpu.sync_copy(acc2, accp_hbm.at[pl.ds(rows[2] * N, N)])
        pltpu.sync_copy(acc3, accp_hbm.at[pl.ds(rows[3] * N, N)])
        plsc.subcore_barrier()

        for k in range(2):
            pltpu.sync_copy(accp_hbm.at[pl.ds(rows[k] * N, N)], den)

            @plsc.parallel_loop(0, N // L, unroll=8)
            def red_g(g):
                accs[k][pl.ds(g * L, L)] = (
                    accs[k][pl.ds(g * L, L)] + den[pl.ds(g * L, L)])
            pltpu.sync_copy(accs[k], acc_hbm.at[pl.ds(rows[k] * N, N)])

    return body(ht.reshape(H * N), a_s, a_d, src, dst)[:3]


def _rating_gather(pt, qt, u_idx, m_idx, *, interpret=False):
    H, N = pt.shape
    B = u_idx.shape[0]
    assert B % L == 0

    mesh = plsc.VectorSubcoreMesh(core_axis_name="c", subcore_axis_name="s", num_cores=NC, num_subcores=NS)

    @functools.partial(
        pl.kernel,
        mesh=mesh,
        out_type=jax.ShapeDtypeStruct((NW * B,), jnp.float32),
        scratch_types=[
            pltpu.VMEM((N,), jnp.float32),
            pltpu.VMEM((N,), jnp.float32),
            pltpu.VMEM((N,), jnp.float32),
            pltpu.VMEM((N,), jnp.float32),
            pltpu.VMEM((B,), jnp.int32),
            pltpu.VMEM((B,), jnp.int32),
            pltpu.VMEM((B,), jnp.float32),
            pltpu.SemaphoreType.DMA,
        ],
        compiler_params=pltpu.CompilerParams(use_tc_tiling_on_sc=False, needs_layout_passes=False),
        interpret=interpret,
    )
    def body(pt_hbm, qt_hbm, u_hbm, m_hbm, out_hbm,
             p0, p1, q0, q1, ubuf, mbuf, obuf, sem):
        c = lax.axis_index("c")
        s = lax.axis_index("s")
        wid = c * NS + s
        f0 = wid * 2

        pltpu.async_copy(pt_hbm.at[pl.ds(f0 * N, N)], p0, sem)
        pltpu.async_copy(pt_hbm.at[pl.ds((f0 + 1) * N, N)], p1, sem)
        pltpu.async_copy(qt_hbm.at[pl.ds(f0 * N, N)], q0, sem)
        pltpu.async_copy(qt_hbm.at[pl.ds((f0 + 1) * N, N)], q1, sem)
        pltpu.async_copy(u_hbm, ubuf, sem)
        pltpu.async_copy(m_hbm, mbuf, sem)
        pltpu.make_async_copy(pt_hbm.at[pl.ds(f0 * N, N)], p0, sem).wait()
        pltpu.make_async_copy(pt_hbm.at[pl.ds(f0 * N, N)], p1, sem).wait()
        pltpu.make_async_copy(qt_hbm.at[pl.ds(f0 * N, N)], q0, sem).wait()
        pltpu.make_async_copy(qt_hbm.at[pl.ds(f0 * N, N)], q1, sem).wait()
        pltpu.make_async_copy(u_hbm, ubuf, sem).wait()
        pltpu.make_async_copy(m_hbm, mbuf, sem).wait()

        @plsc.parallel_loop(0, B // L, unroll=8)
        def gbody(g):
            u16 = ubuf[pl.ds(g * L, L)]
            m16 = mbuf[pl.ds(g * L, L)]
            r = (plsc.load_gather(p0, [u16]) + plsc.load_gather(p1, [u16])
                 + plsc.load_gather(q0, [m16]) + plsc.load_gather(q1, [m16]))
            obuf[pl.ds(g * L, L)] = r
        pltpu.sync_copy(obuf, out_hbm.at[pl.ds(wid * B, B)])

    return body(pt.reshape(H * N), qt.reshape(H * N), u_idx, m_idx)


def _tc1_body(x_ref, w1t_ref, a1s_ref, a1d_ref, ht_ref, as_ref, ad_ref):
    ht = lax.dot_general(w1t_ref[...], x_ref[...],
                         (((1,), (1,)), ((), ())),
                         preferred_element_type=jnp.float32)
    ht_ref[...] = ht
    as_ref[...] = jnp.dot(a1s_ref[...], ht, preferred_element_type=jnp.float32)
    ad_ref[...] = jnp.dot(a1d_ref[...], ht, preferred_element_type=jnp.float32)


def _tc1(x, w1t, a1s, a1d):
    N = x.shape[0]
    H = w1t.shape[0]
    return pl.pallas_call(
        _tc1_body,
        out_shape=(
            jax.ShapeDtypeStruct((H, N), jnp.float32),
            jax.ShapeDtypeStruct((1, N), jnp.float32),
            jax.ShapeDtypeStruct((1, N), jnp.float32),
        ),
    )(x, w1t, a1s, a1d)


def _norm_next(accT, denP, ht, as_v, ad_v, b_col, w2t, a2s, a2d):
    def body(acc_ref, den_ref, ht_ref, as_ref, ad_ref, b_ref,
             w2t_ref, a2s_ref, a2d_ref, h2_ref, as2_ref, ad2_ref):
        e = as_ref[...] + ad_ref[...]
        wself = jnp.exp(jnp.where(e >= 0.0, e, 0.2 * e))
        den = jnp.sum(den_ref[...][:NS, :], axis=0, keepdims=True) + wself
        num = acc_ref[...] + wself * ht_ref[...]
        out1 = num / (den + 1e-16) + b_ref[...]
        r = jnp.maximum(out1, 0.0)
        h2 = lax.dot_general(w2t_ref[...], r, (((1,), (0,)), ((), ())),
                             preferred_element_type=jnp.float32)
        h2_ref[...] = h2
        as2_ref[...] = jnp.dot(a2s_ref[...], h2,
                               preferred_element_type=jnp.float32)
        ad2_ref[...] = jnp.dot(a2d_ref[...], h2,
                               preferred_element_type=jnp.float32)

    H, N = ht.shape
    return pl.pallas_call(
        body,
        out_shape=(
            jax.ShapeDtypeStruct((H, N), jnp.float32),
            jax.ShapeDtypeStruct((1, N), jnp.float32),
            jax.ShapeDtypeStruct((1, N), jnp.float32),
        ),
    )(accT, denP, ht, as_v, ad_v, b_col, w2t, a2s, a2d)


def _norm_scale(accT, denP, ht, as_v, ad_v, b_col, fwu, fwm):
    def body(acc_ref, den_ref, ht_ref, as_ref, ad_ref, b_ref,
             fwu_ref, fwm_ref, pt_ref, qt_ref):
        e = as_ref[...] + ad_ref[...]
        wself = jnp.exp(jnp.where(e >= 0.0, e, 0.2 * e))
        den = jnp.sum(den_ref[...][:NS, :], axis=0, keepdims=True) + wself
        num = acc_ref[...] + wself * ht_ref[...]
        out2 = num / (den + 1e-16) + b_ref[...]
        pt_ref[...] = out2 * fwu_ref[...]
        qt_ref[...] = out2 * fwm_ref[...]

    H, N = ht.shape
    return pl.pallas_call(
        body,
        out_shape=(
            jax.ShapeDtypeStruct((H, N), jnp.float32),
            jax.ShapeDtypeStruct((H, N), jnp.float32),
        ),
    )(accT, denP, ht, as_v, ad_v, b_col, fwu, fwm)


def _reduce_partials(partial, fcb):
    def body(p_ref, b_ref, o_ref):
        o_ref[...] = jnp.sum(p_ref[...], axis=0, keepdims=True) + b_ref[...]

    NWp, B = partial.shape
    return pl.pallas_call(
        body,
        out_shape=jax.ShapeDtypeStruct((1, B), jnp.float32),
    )(partial, fcb)


def kernel(x, edge_index, user_indices, movie_indices,
           W1, a1_src, a1_dst, b1, W2, a2_src, a2_dst, b2, fc_W, fc_b):
    N, F_in = x.shape
    H = W1.shape[1]
    E = edge_index.shape[1]
    B = user_indices.shape[0]

    src = edge_index[0]
    dst = edge_index[1]

    h1t, as1, ad1 = _tc1(x, W1.T, a1_src.reshape(1, H), a1_dst.reshape(1, H))
    acc1, denp1, _ = _edge_pass(h1t, as1.reshape(N), ad1.reshape(N), src, dst)
    acc1 = acc1.reshape(H, N)
    denp1 = denp1.reshape(NW, N)

    h2t, as2, ad2 = _norm_next(acc1, denp1, h1t, as1, ad1, b1.reshape(H, 1),
                               W2.T, a2_src.reshape(1, H), a2_dst.reshape(1, H))
    acc2, denp2, _ = _edge_pass(h2t, as2.reshape(N), ad2.reshape(N), src, dst)
    acc2 = acc2.reshape(H, N)
    denp2 = denp2.reshape(NW, N)

    pt, qt = _norm_scale(acc2, denp2, h2t, as2, ad2, b2.reshape(H, 1),
                         fc_W[:H].reshape(H, 1), fc_W[H:].reshape(H, 1))

    partial = _rating_gather(pt, qt, user_indices, movie_indices)
    out = _reduce_partials(partial.reshape(NW, B), fc_b.reshape(1, 1))
    return out.reshape(B, 1)


# speedup vs baseline: 1.2800x; 1.0332x over previous
import functools

import jax
import jax.numpy as jnp
from jax import lax
from jax.experimental import pallas as pl
from jax.experimental.pallas import tpu as pltpu
from jax.experimental.pallas import tpu_sc as plsc

NC = 2
NS = 16
L = 16
NW = NC * NS


def _edge_pass(ht, a_s, a_d, src, dst, sd=None, *, interpret=False):
    H, N = ht.shape
    E = src.shape[0]
    has_sd = sd is not None
    assert N <= (1 << 14)
    EH = E // 2
    CH1 = 2000 if (E // NS) % 2000 == 0 else E // NS
    CH = 8000 if EH % (2 * 8000) == 0 else E // NS
    assert H == NC * (NS // 2) * 4 and NS % 2 == 0
    assert (E // NS) % CH1 == 0 and CH1 % L == 0 and N % L == 0
    assert EH % CH == 0 and (EH // CH) % 2 == 0 and CH % L == 0 and CH >= CH1

    mesh = plsc.VectorSubcoreMesh(core_axis_name="c", subcore_axis_name="s", num_cores=NC, num_subcores=NS)

    out_type = [
        jax.ShapeDtypeStruct((H * N,), jnp.float32),
        jax.ShapeDtypeStruct((NW * N,), jnp.float32),
        jax.ShapeDtypeStruct((NC * E,), jnp.float32),
        jax.ShapeDtypeStruct((H * N,), jnp.float32),
    ]
    if not has_sd:
        out_type.append(jax.ShapeDtypeStruct((NC * E,), jnp.int32))

    @functools.partial(
        pl.kernel,
        mesh=mesh,
        out_type=tuple(out_type),
        scratch_types=[
            pltpu.VMEM((N,), jnp.float32),
            pltpu.VMEM((N,), jnp.float32),
            pltpu.VMEM((N,), jnp.float32),
            pltpu.VMEM((N,), jnp.float32),
            pltpu.VMEM((N,), jnp.float32),
            pltpu.VMEM((N,), jnp.float32),
            pltpu.VMEM((N,), jnp.float32),
            pltpu.VMEM((N,), jnp.float32),
            pltpu.VMEM((N,), jnp.float32),
            pltpu.VMEM((CH,), jnp.int32),
            pltpu.VMEM((CH1,), jnp.int32),
            pltpu.VMEM((CH,), jnp.float32),
            pltpu.VMEM((CH,), jnp.int32),
            pltpu.VMEM((CH1,), jnp.int32),
            pltpu.VMEM((CH,), jnp.float32),
            pltpu.SemaphoreType.DMA,
            pltpu.SemaphoreType.DMA,
            pltpu.SemaphoreType.DMA,
            pltpu.SemaphoreType.DMA,
        ],
        compiler_params=pltpu.CompilerParams(use_tc_tiling_on_sc=False, needs_layout_passes=False),
        interpret=interpret,
    )
    def body(*refs):
        if has_sd:
            (ht_hbm, as_hbm, ad_hbm, sd_hbm,
             acc_hbm, denp_hbm, w_hbm, accp_hbm,
             h0, h1, t0, t1, acc0, acc1, acc2, acc3, den,
             sbuf, dbuf, wbuf, sbuf1, dbuf1, wbuf1,
             sem0, sem1, semw0, semw1) = refs
            src_hbm = dst_hbm = sdout_hbm = None
        else:
            (ht_hbm, as_hbm, ad_hbm, src_hbm, dst_hbm,
             acc_hbm, denp_hbm, w_hbm, accp_hbm, sdout_hbm,
             h0, h1, t0, t1, acc0, acc1, acc2, acc3, den,
             sbuf, dbuf, wbuf, sbuf1, dbuf1, wbuf1,
             sem0, sem1, semw0, semw1) = refs
            sd_hbm = sdout_hbm
        c = lax.axis_index("c")
        s = lax.axis_index("s")
        wid = c * NS + s
        half = s % 2
        base_f = (c * (NS // 2) + s // 2) * 4
        rows = [base_f + (k + 2 * half) % 4 for k in range(4)]

        pltpu.sync_copy(as_hbm, t0)
        pltpu.sync_copy(ad_hbm, t1)

        zero16 = jnp.zeros((L,), jnp.float32)

        def zbody(i, _):
            acc0[pl.ds(i * L, L)] = zero16
            acc1[pl.ds(i * L, L)] = zero16
            acc2[pl.ds(i * L, L)] = zero16
            acc3[pl.ds(i * L, L)] = zero16
            den[pl.ds(i * L, L)] = zero16
            return _
        lax.fori_loop(0, N // L, zbody, None)

        base_s = s * (E // NS)
        nch1 = (E // NS) // CH1
        assert nch1 % 2 == 0

        def p1_issue(t, sb, db, sem):
            base = base_s + t * CH1
            if has_sd:
                pltpu.async_copy(sd_hbm.at[pl.ds(c * E + base, CH1)],
                                 sb.at[pl.ds(0, CH1)], sem)
            else:
                pltpu.async_copy(src_hbm.at[pl.ds(base, CH1)],
                                 sb.at[pl.ds(0, CH1)], sem)
                pltpu.async_copy(dst_hbm.at[pl.ds(base, CH1)], db, sem)

        def p1_drain(t, sb, db, sem):
            base = base_s + t * CH1
            if has_sd:
                pltpu.make_async_copy(sd_hbm.at[pl.ds(c * E + base, CH1)],
                                      sb.at[pl.ds(0, CH1)], sem).wait()
            else:
                pltpu.make_async_copy(src_hbm.at[pl.ds(base, CH1)],
                                      sb.at[pl.ds(0, CH1)], sem).wait()
                pltpu.make_async_copy(dst_hbm.at[pl.ds(base, CH1)],
                                      db, sem).wait()

        def p1_compute(t, sb, db, wb, wsem):
            if has_sd:
                @plsc.parallel_loop(0, CH1 // L, unroll=4)
                def p1_g(g):
                    sd16 = sb[pl.ds(g * L, L)]
                    s16 = sd16 & 0x3FFF
                    d16 = sd16 >> 14
                    e = (plsc.load_gather(t0, [s16])
                         + plsc.load_gather(t1, [d16]))
                    e = jnp.where(e >= 0.0, e, 0.2 * e)
                    w = jnp.exp(e)
                    wb[pl.ds(g * L, L)] = w
                    plsc.addupdate_scatter(den, [d16], w)
            else:
                @plsc.parallel_loop(0, CH1 // L, unroll=4)
                def p1_g(g):
                    s16 = sb[pl.ds(g * L, L)]
                    d16 = db[pl.ds(g * L, L)]
                    e = (plsc.load_gather(t0, [s16])
                         + plsc.load_gather(t1, [d16]))
                    e = jnp.where(e >= 0.0, e, 0.2 * e)
                    w = jnp.exp(e)
                    wb[pl.ds(g * L, L)] = w
                    plsc.addupdate_scatter(den, [d16], w)
                    sb[pl.ds(g * L, L)] = s16 | (d16 << 14)
            base = base_s + t * CH1
            pltpu.async_copy(wb.at[pl.ds(0, CH1)],
                             w_hbm.at[pl.ds(c * E + base, CH1)], wsem)
            if not has_sd:
                pltpu.async_copy(sb.at[pl.ds(0, CH1)],
                                 sd_hbm.at[pl.ds(c * E + base, CH1)], wsem)

        def p1_wdrain(t, sb, wb, wsem):
            base = base_s + t * CH1
            pltpu.make_async_copy(
                wb.at[pl.ds(0, CH1)],
                w_hbm.at[pl.ds(c * E + base, CH1)], wsem).wait()
            if not has_sd:
                pltpu.make_async_copy(
                    sb.at[pl.ds(0, CH1)],
                    sd_hbm.at[pl.ds(c * E + base, CH1)], wsem).wait()

        p1_issue(0, sbuf, dbuf, sem0)

        def p1_pair(t, carry):
            c0 = 2 * t
            p1_drain(c0, sbuf, dbuf, sem0)

            @pl.when(t > 0)
            def _wd1():
                p1_wdrain(c0 - 1, sbuf1, wbuf1, semw1)
            p1_issue(c0 + 1, sbuf1, dbuf1, sem1)
            p1_compute(c0, sbuf, dbuf, wbuf, semw0)
            p1_drain(c0 + 1, sbuf1, dbuf1, sem1)

            @pl.when(t + 1 < nch1 // 2)
            def _issue_next():
                p1_wdrain(c0, sbuf, wbuf, semw0)
                p1_issue(c0 + 2, sbuf, dbuf, sem0)
            p1_compute(c0 + 1, sbuf1, dbuf1, wbuf1, semw1)
            return carry
        lax.fori_loop(0, nch1 // 2, p1_pair, None)
        p1_wdrain(nch1 - 2, sbuf, wbuf, semw0)
        p1_wdrain(nch1 - 1, sbuf1, wbuf1, semw1)
        pltpu.sync_copy(den, denp_hbm.at[pl.ds(wid * N, N)])
        plsc.subcore_barrier()

        accs = (acc0, acc1, acc2, acc3)
        for k, hp in ((0, t0), (2, t1)):
            pltpu.sync_copy(ht_hbm.at[pl.ds(rows[k] * N, N)], h0)
            pltpu.sync_copy(ht_hbm.at[pl.ds(rows[k + 1] * N, N)], h1)

            @plsc.parallel_loop(0, N // L, unroll=8)
            def pk_g(g):
                a = h0[pl.ds(g * L, L)]
                b = h1[pl.ds(g * L, L)]
                packed = plsc.pack(a, b, format=plsc.PackFormat.INTERLEAVED)
                hp[pl.ds(g * L, L)] = plsc.bitcast(packed, jnp.float32)

        bo = half * EH

        def issue(chunk, sb, wb, sem):
            base = bo + chunk * CH
            pltpu.async_copy(sd_hbm.at[pl.ds(c * E + base, CH)], sb, sem)
            pltpu.async_copy(w_hbm.at[pl.ds(c * E + base, CH)], wb, sem)

        def drain(chunk, sb, wb, sem):
            base = bo + chunk * CH
            pltpu.make_async_copy(
                sd_hbm.at[pl.ds(c * E + base, CH)], sb, sem).wait()
            pltpu.make_async_copy(
                w_hbm.at[pl.ds(c * E + base, CH)], wb, sem).wait()

        def compute(sb, wb):
            @plsc.parallel_loop(0, CH // L, unroll=8)
            def p2_g(g):
                sd16 = sb[pl.ds(g * L, L)]
                s16 = sd16 & 0x3FFF
                d16 = sd16 >> 14
                w16 = wb[pl.ds(g * L, L)]
                for k, hp in ((0, t0), (2, t1)):
                    g32 = plsc.load_gather(hp, [s16])
                    ab = plsc.bitcast(g32, jnp.bfloat16)
                    va, vb = plsc.unpack(
                        ab, format=plsc.PackFormat.INTERLEAVED)
                    plsc.addupdate_scatter(accs[k], [d16], va * w16)
                    plsc.addupdate_scatter(accs[k + 1], [d16], vb * w16)

        nch = EH // CH
        issue(0, sbuf, wbuf, sem0)

        def p2_pair(t, carry):
            c0 = 2 * t
            drain(c0, sbuf, wbuf, sem0)
            issue(c0 + 1, sbuf1, wbuf1, sem1)
            compute(sbuf, wbuf)
            drain(c0 + 1, sbuf1, wbuf1, sem1)

            @pl.when(t + 1 < nch // 2)
            def _issue_next():
                issue(c0 + 2, sbuf, wbuf, sem0)
            compute(sbuf1, wbuf1)
            return carry
        lax.fori_loop(0, nch // 2, p2_pair, None)

        pltpu.sync_copy(acc2, accp_hbm.at[pl.ds(rows[2] * N, N)])
        pltpu.sync_copy(acc3, accp_hbm.at[pl.ds(rows[3] * N, N)])
        plsc.subcore_barrier()

        for k in range(2):
            pltpu.sync_copy(accp_hbm.at[pl.ds(rows[k] * N, N)], den)

            @plsc.parallel_loop(0, N // L, unroll=8)
            def red_g(g):
                accs[k][pl.ds(g * L, L)] = (
                    accs[k][pl.ds(g * L, L)] + den[pl.ds(g * L, L)])
            pltpu.sync_copy(accs[k], acc_hbm.at[pl.ds(rows[k] * N, N)])

    if has_sd:
        acc, denp, _, _ = body(ht.reshape(H * N), a_s, a_d, sd)
        return acc, denp, None
    acc, denp, _, _, sd_out = body(ht.reshape(H * N), a_s, a_d, src, dst)
    return acc, denp, sd_out


def _rating_gather(pt, qt, u_idx, m_idx, *, interpret=False):
    H, N = pt.shape
    B = u_idx.shape[0]
    assert B % L == 0

    mesh = plsc.VectorSubcoreMesh(core_axis_name="c", subcore_axis_name="s", num_cores=NC, num_subcores=NS)

    @functools.partial(
        pl.kernel,
        mesh=mesh,
        out_type=jax.ShapeDtypeStruct((NW * B,), jnp.float32),
        scratch_types=[
            pltpu.VMEM((N,), jnp.float32),
            pltpu.VMEM((N,), jnp.float32),
            pltpu.VMEM((N,), jnp.float32),
            pltpu.VMEM((N,), jnp.float32),
            pltpu.VMEM((B,), jnp.int32),
            pltpu.VMEM((B,), jnp.int32),
            pltpu.VMEM((B,), jnp.float32),
            pltpu.SemaphoreType.DMA,
        ],
        compiler_params=pltpu.CompilerParams(use_tc_tiling_on_sc=False, needs_layout_passes=False),
        interpret=interpret,
    )
    def body(pt_hbm, qt_hbm, u_hbm, m_hbm, out_hbm,
             p0, p1, q0, q1, ubuf, mbuf, obuf, sem):
        c = lax.axis_index("c")
        s = lax.axis_index("s")
        wid = c * NS + s
        f0 = wid * 2

        pltpu.async_copy(pt_hbm.at[pl.ds(f0 * N, N)], p0, sem)
        pltpu.async_copy(pt_hbm.at[pl.ds((f0 + 1) * N, N)], p1, sem)
        pltpu.async_copy(qt_hbm.at[pl.ds(f0 * N, N)], q0, sem)
        pltpu.async_copy(qt_hbm.at[pl.ds((f0 + 1) * N, N)], q1, sem)
        pltpu.async_copy(u_hbm, ubuf, sem)
        pltpu.async_copy(m_hbm, mbuf, sem)
        pltpu.make_async_copy(pt_hbm.at[pl.ds(f0 * N, N)], p0, sem).wait()
        pltpu.make_async_copy(pt_hbm.at[pl.ds(f0 * N, N)], p1, sem).wait()
        pltpu.make_async_copy(qt_hbm.at[pl.ds(f0 * N, N)], q0, sem).wait()
        pltpu.make_async_copy(qt_hbm.at[pl.ds(f0 * N, N)], q1, sem).wait()
        pltpu.make_async_copy(u_hbm, ubuf, sem).wait()
        pltpu.make_async_copy(m_hbm, mbuf, sem).wait()

        @plsc.parallel_loop(0, B // L, unroll=8)
        def gbody(g):
            u16 = ubuf[pl.ds(g * L, L)]
            m16 = mbuf[pl.ds(g * L, L)]
            r = (plsc.load_gather(p0, [u16]) + plsc.load_gather(p1, [u16])
                 + plsc.load_gather(q0, [m16]) + plsc.load_gather(q1, [m16]))
            obuf[pl.ds(g * L, L)] = r
        pltpu.sync_copy(obuf, out_hbm.at[pl.ds(wid * B, B)])

    return body(pt.reshape(H * N), qt.reshape(H * N), u_idx, m_idx)


def _tc1_body(x_ref, w1t_ref, a1s_ref, a1d_ref, ht_ref, as_ref, ad_ref):
    ht = lax.dot_general(w1t_ref[...], x_ref[...],
                         (((1,), (1,)), ((), ())),
                         preferred_element_type=jnp.float32)
    ht_ref[...] = ht
    as_ref[...] = jnp.dot(a1s_ref[...], ht, preferred_element_type=jnp.float32)
    ad_ref[...] = jnp.dot(a1d_ref[...], ht, preferred_element_type=jnp.float32)


def _tc1(x, w1t, a1s, a1d):
    N = x.shape[0]
    H = w1t.shape[0]
    return pl.pallas_call(
        _tc1_body,
        out_shape=(
            jax.ShapeDtypeStruct((H, N), jnp.float32),
            jax.ShapeDtypeStruct((1, N), jnp.float32),
            jax.ShapeDtypeStruct((1, N), jnp.float32),
        ),
    )(x, w1t, a1s, a1d)


def _norm_next(accT, denP, ht, as_v, ad_v, b_col, w2t, a2s, a2d):
    def body(acc_ref, den_ref, ht_ref, as_ref, ad_ref, b_ref,
             w2t_ref, a2s_ref, a2d_ref, h2_ref, as2_ref, ad2_ref):
        e = as_ref[...] + ad_ref[...]
        wself = jnp.exp(jnp.where(e >= 0.0, e, 0.2 * e))
        den = jnp.sum(den_ref[...][:NS, :], axis=0, keepdims=True) + wself
        num = acc_ref[...] + wself * ht_ref[...]
        out1 = num / (den + 1e-16) + b_ref[...]
        r = jnp.maximum(out1, 0.0)
        h2 = lax.dot_general(w2t_ref[...], r, (((1,), (0,)), ((), ())),
                             preferred_element_type=jnp.float32)
        h2_ref[...] = h2
        as2_ref[...] = jnp.dot(a2s_ref[...], h2,
                               preferred_element_type=jnp.float32)
        ad2_ref[...] = jnp.dot(a2d_ref[...], h2,
                               preferred_element_type=jnp.float32)

    H, N = ht.shape
    return pl.pallas_call(
        body,
        out_shape=(
            jax.ShapeDtypeStruct((H, N), jnp.float32),
            jax.ShapeDtypeStruct((1, N), jnp.float32),
            jax.ShapeDtypeStruct((1, N), jnp.float32),
        ),
    )(accT, denP, ht, as_v, ad_v, b_col, w2t, a2s, a2d)


def _norm_scale(accT, denP, ht, as_v, ad_v, b_col, fwu, fwm):
    def body(acc_ref, den_ref, ht_ref, as_ref, ad_ref, b_ref,
             fwu_ref, fwm_ref, pt_ref, qt_ref):
        e = as_ref[...] + ad_ref[...]
        wself = jnp.exp(jnp.where(e >= 0.0, e, 0.2 * e))
        den = jnp.sum(den_ref[...][:NS, :], axis=0, keepdims=True) + wself
        num = acc_ref[...] + wself * ht_ref[...]
        out2 = num / (den + 1e-16) + b_ref[...]
        pt_ref[...] = out2 * fwu_ref[...]
        qt_ref[...] = out2 * fwm_ref[...]

    H, N = ht.shape
    return pl.pallas_call(
        body,
        out_shape=(
            jax.ShapeDtypeStruct((H, N), jnp.float32),
            jax.ShapeDtypeStruct((H, N), jnp.float32),
        ),
    )(accT, denP, ht, as_v, ad_v, b_col, fwu, fwm)


def _reduce_partials(partial, fcb):
    def body(p_ref, b_ref, o_ref):
        o_ref[...] = jnp.sum(p_ref[...], axis=0, keepdims=True) + b_ref[...]

    NWp, B = partial.shape
    return pl.pallas_call(
        body,
        out_shape=jax.ShapeDtypeStruct((1, B), jnp.float32),
    )(partial, fcb)


def kernel(x, edge_index, user_indices, movie_indices,
           W1, a1_src, a1_dst, b1, W2, a2_src, a2_dst, b2, fc_W, fc_b):
    N, F_in = x.shape
    H = W1.shape[1]
    E = edge_index.shape[1]
    B = user_indices.shape[0]

    src = edge_index[0]
    dst = edge_index[1]

    h1t, as1, ad1 = _tc1(x, W1.T, a1_src.reshape(1, H), a1_dst.reshape(1, H))
    acc1, denp1, sd1 = _edge_pass(h1t, as1.reshape(N), ad1.reshape(N),
                                  src, dst)
    acc1 = acc1.reshape(H, N)
    denp1 = denp1.reshape(NW, N)

    h2t, as2, ad2 = _norm_next(acc1, denp1, h1t, as1, ad1, b1.reshape(H, 1),
                               W2.T, a2_src.reshape(1, H), a2_dst.reshape(1, H))
    acc2, denp2, _ = _edge_pass(h2t, as2.reshape(N), ad2.reshape(N),
                                src, dst, sd=sd1)
    acc2 = acc2.reshape(H, N)
    denp2 = denp2.reshape(NW, N)

    pt, qt = _norm_scale(acc2, denp2, h2t, as2, ad2, b2.reshape(H, 1),
                         fc_W[:H].reshape(H, 1), fc_W[H:].reshape(H, 1))

    partial = _rating_gather(pt, qt, user_indices, movie_indices)
    out = _reduce_partials(partial.reshape(NW, B), fc_b.reshape(1, 1))
    return out.reshape(B, 1)
